# baseline (speedup 1.0000x reference)
import functools

import jax
import jax.numpy as jnp
from jax import lax
from jax.experimental import pallas as pl
from jax.experimental.pallas import tpu as pltpu
from jax.experimental.pallas import tpu_sc as plsc

_LANES = 16
_CHUNK = 128
_REPACK_W = 8192
_SH_BLK = _REPACK_W.bit_length() - 1
_SH_HALF = _SH_BLK - 1
_HALF_MASK = (1 << _SH_HALF) - 1


@functools.lru_cache(maxsize=None)
def _build(n_tokens: int, seq_len: int, dim: int):
    info = plsc.get_sparse_core_info()
    n_workers = info.num_cores * info.num_subcores
    assert n_tokens % (n_workers * 2 * _CHUNK) == 0
    per_worker = n_tokens // n_workers
    n_chunks = per_worker // _CHUNK
    n_groups = n_chunks // 2
    nblk = dim // _LANES
    inv_dim = 1.0 / dim
    mesh = plsc.VectorSubcoreMesh(core_axis_name="c", subcore_axis_name="s")

    def body(ids_hbm, tt_hbm, tok_hbm, pos_hbm, seg_hbm, g_hbm, b_hbm,
             out_hbm, pos_v, seg_v, g_v, b_v, raw_all, tt_all,
             idx0, idx1, rows0, rows1, outv0, outv1,
             semg0, semg1, semo0, semo1):
        idx = (idx0, idx1)
        rows = (rows0, rows1)
        outv = (outv0, outv1)
        semg = (semg0, semg1)
        semo = (semo0, semo1)
        wid = lax.axis_index("s") * info.num_cores + lax.axis_index("c")
        w_base = wid * per_worker
        pltpu.sync_copy(pos_hbm, pos_v)
        pltpu.sync_copy(seg_hbm, seg_v)
        pltpu.sync_copy(g_hbm, g_v)
        pltpu.sync_copy(b_hbm, b_v)
        pltpu.sync_copy(ids_hbm.at[pl.ds(w_base, per_worker)],
                        raw_all.at[pl.ds(0, per_worker)])
        pltpu.sync_copy(tt_hbm.at[pl.ds(w_base, per_worker)],
                        tt_all.at[pl.ds(0, per_worker)])
        seg0 = [seg_v[0, pl.ds(i * _LANES, _LANES)] for i in range(nblk)]
        segd = [seg_v[1, pl.ds(i * _LANES, _LANES)] - seg0[i] for i in range(nblk)]
        gam = [g_v[pl.ds(i * _LANES, _LANES)] for i in range(nblk)]
        bet = [b_v[pl.ds(i * _LANES, _LANES)] for i in range(nblk)]

        def build_idx(c, b):
            coff = c * _CHUNK
            for i in range(_CHUNK // _LANES):
                v = raw_all[pl.ds(coff + i * _LANES, _LANES)]
                idx[b][pl.ds(i * _LANES, _LANES)] = (
                    lax.shift_left(lax.shift_right_logical(v, _SH_BLK),
                                   _SH_HALF)
                    | (v & _HALF_MASK))

        def tok_body(c, rows_b, out_b, t):
            coff = c * _CHUNK
            s = lax.rem(w_base + coff + t, seq_len)
            off = (lax.shift_right_logical(
                raw_all[pl.ds(coff + t, _LANES)][0], _SH_HALF) & 1) * dim
            ttf = tt_all[pl.ds(coff + t, _LANES)][0].astype(jnp.float32)
            e = []
            for i in range(nblk):
                w = rows_b[t, pl.ds(off + i * _LANES, _LANES)]
                p = pos_v[s, pl.ds(i * _LANES, _LANES)]
                e.append((w + p) + (seg0[i] + ttf * segd[i]))
            s1 = (e[0] + e[1]) + (e[2] + e[3])
            s2 = (e[0] * e[0] + e[1] * e[1]) + (e[2] * e[2] + e[3] * e[3])
            mean = jnp.sum(s1) * inv_dim
            x = jnp.sum(s2) * inv_dim - mean * mean + 1e-12
            ib = 0x5F3759DF - lax.shift_right_logical(
                lax.bitcast_convert_type(x, jnp.int32), 1)
            y = lax.bitcast_convert_type(ib, jnp.float32)
            for _ in range(3):
                y = y * (1.5 - 0.5 * x * y * y)
            for i in range(nblk):
                out_b[t, pl.ds(i * _LANES, _LANES)] = (
                    (e[i] - mean) * (y * gam[i]) + bet[i])

        build_idx(0, 0)
        pltpu.async_copy(tok_hbm.at[idx[0]], rows[0], semg[0]).wait()

        def group_body(g, _):
            for b in (0, 1):
                c = 2 * g + b
                nb = 1 - b
                base = w_base + c * _CHUNK
                have_next = (g < n_groups - 1) if b else True

                def fire_next():
                    build_idx(c + 1, nb)
                    pltpu.async_copy(tok_hbm.at[idx[nb]], rows[nb], semg[nb])

                if b == 0:
                    fire_next()
                else:
                    pl.when(g < n_groups - 1)(fire_next)

                @pl.when(g >= 1)
                def _():
                    pltpu.make_async_copy(
                        outv[b],
                        out_hbm.at[pl.ds(base - 2 * _CHUNK, _CHUNK)],
                        semo[b]).wait()

                plsc.parallel_loop(0, _CHUNK, unroll=8)(
                    functools.partial(tok_body, c, rows[b], outv[b]))
                pltpu.async_copy(outv[b], out_hbm.at[pl.ds(base, _CHUNK)],
                                 semo[b])

                def wait_next():
                    pltpu.make_async_copy(tok_hbm.at[idx[nb]], rows[nb],
                                          semg[nb]).wait()

                if b == 0:
                    wait_next()
                else:
                    pl.when(g < n_groups - 1)(wait_next)
            return 0

        lax.fori_loop(0, n_groups, group_body, 0)
        pltpu.make_async_copy(
            outv[0], out_hbm.at[pl.ds(w_base + (n_chunks - 2) * _CHUNK, _CHUNK)],
            semo[0]).wait()
        pltpu.make_async_copy(
            outv[1], out_hbm.at[pl.ds(w_base + (n_chunks - 1) * _CHUNK, _CHUNK)],
            semo[1]).wait()

    return pl.kernel(
        body,
        out_type=jax.ShapeDtypeStruct((n_tokens, dim), jnp.float32),
        mesh=mesh,
        compiler_params=pltpu.CompilerParams(needs_layout_passes=False),
        scratch_types=[
            pltpu.VMEM((seq_len, dim), jnp.float32),
            pltpu.VMEM((2, dim), jnp.float32),
            pltpu.VMEM((dim,), jnp.float32),
            pltpu.VMEM((dim,), jnp.float32),
            pltpu.VMEM((per_worker + _LANES,), jnp.int32),
            pltpu.VMEM((per_worker + _LANES,), jnp.int32),
            pltpu.VMEM((_CHUNK,), jnp.int32),
            pltpu.VMEM((_CHUNK,), jnp.int32),
            pltpu.VMEM((_CHUNK, 2 * dim), jnp.float32),
            pltpu.VMEM((_CHUNK, 2 * dim), jnp.float32),
            pltpu.VMEM((_CHUNK, dim), jnp.float32),
            pltpu.VMEM((_CHUNK, dim), jnp.float32),
            pltpu.SemaphoreType.DMA,
            pltpu.SemaphoreType.DMA,
            pltpu.SemaphoreType.DMA,
            pltpu.SemaphoreType.DMA,
        ],
    )


@functools.lru_cache(maxsize=None)
def _build_tc_repack(vocab: int, dim: int):
    w = _REPACK_W
    n_blocks = (vocab + w - 1) // w

    def tkernel(x_ref, o_ref):
        x = x_ref[...]
        xt = jnp.swapaxes(x, 0, 1)
        d = x.shape[0]
        o_ref[:, 0:d] = xt[0:w // 2]
        o_ref[:, d:2 * d] = xt[w // 2:w]

    return pl.pallas_call(
        tkernel,
        grid=(n_blocks,),
        in_specs=[pl.BlockSpec((dim, w), lambda i: (0, i))],
        out_specs=pl.BlockSpec((w // 2, 2 * dim), lambda i: (i, 0)),
        out_shape=jax.ShapeDtypeStruct((n_blocks * (w // 2), 2 * dim),
                                       jnp.float32),
    )


def kernel(input_ids, token_type_ids, token_table, pos_table, seg_table,
           ln_gamma, ln_beta):
    bsz, seq = input_ids.shape
    vocab, dim = token_table.shape
    n = bsz * seq
    ids = input_ids.reshape(n).astype(jnp.int32)
    tts = token_type_ids.reshape(n).astype(jnp.int32)
    tok2 = _build_tc_repack(vocab, dim)(token_table.T)
    out = _build(n, seq, dim)(ids, tts, tok2, pos_table[:seq],
                              seg_table, ln_gamma, ln_beta)
    return out.reshape(bsz, seq, dim)


# speedup vs baseline: 1.0707x; 1.0707x over previous
import functools

import jax
import jax.numpy as jnp
from jax import lax
from jax.experimental import pallas as pl
from jax.experimental.pallas import tpu as pltpu
from jax.experimental.pallas import tpu_sc as plsc

_LANES = 16
_CHUNK = 128
_REPACK_W = 16384
_SH_BLK = _REPACK_W.bit_length() - 1
_SH_HALF = _SH_BLK - 1
_HALF_MASK = (1 << _SH_HALF) - 1


@functools.lru_cache(maxsize=None)
def _build(n_tokens: int, seq_len: int, dim: int):
    info = plsc.get_sparse_core_info()
    n_workers = info.num_cores * info.num_subcores
    assert n_tokens % (n_workers * 2 * _CHUNK) == 0
    per_worker = n_tokens // n_workers
    n_chunks = per_worker // _CHUNK
    n_groups = n_chunks // 2
    nblk = dim // _LANES
    inv_dim = 1.0 / dim
    mesh = plsc.VectorSubcoreMesh(core_axis_name="c", subcore_axis_name="s")

    def body(ids_hbm, tt_hbm, tok_hbm, pos_hbm, seg_hbm, g_hbm, b_hbm,
             out_hbm, pos_v, seg_v, g_v, b_v, raw_all, tt_all,
             idx0, idx1, rows0, rows1, outv0, outv1,
             semg0, semg1, semo0, semo1):
        idx = (idx0, idx1)
        rows = (rows0, rows1)
        outv = (outv0, outv1)
        semg = (semg0, semg1)
        semo = (semo0, semo1)
        wid = lax.axis_index("s") * info.num_cores + lax.axis_index("c")
        w_base = wid * per_worker
        pltpu.sync_copy(pos_hbm, pos_v)
        pltpu.sync_copy(seg_hbm, seg_v)
        pltpu.sync_copy(g_hbm, g_v)
        pltpu.sync_copy(b_hbm, b_v)
        pltpu.sync_copy(ids_hbm.at[pl.ds(w_base, per_worker)],
                        raw_all.at[pl.ds(0, per_worker)])
        pltpu.sync_copy(tt_hbm.at[pl.ds(w_base, per_worker)],
                        tt_all.at[pl.ds(0, per_worker)])
        seg0 = [seg_v[0, pl.ds(i * _LANES, _LANES)] for i in range(nblk)]
        segd = [seg_v[1, pl.ds(i * _LANES, _LANES)] - seg0[i] for i in range(nblk)]
        gam = [g_v[pl.ds(i * _LANES, _LANES)] for i in range(nblk)]
        bet = [b_v[pl.ds(i * _LANES, _LANES)] for i in range(nblk)]

        def build_idx(c, b):
            coff = c * _CHUNK
            for i in range(_CHUNK // _LANES):
                v = raw_all[pl.ds(coff + i * _LANES, _LANES)]
                idx[b][pl.ds(i * _LANES, _LANES)] = (
                    lax.shift_left(lax.shift_right_logical(v, _SH_BLK),
                                   _SH_HALF)
                    | (v & _HALF_MASK))

        def tok_body(c, rows_b, out_b, t):
            coff = c * _CHUNK
            s = lax.rem(w_base + coff + t, seq_len)
            off = (lax.shift_right_logical(
                raw_all[pl.ds(coff + t, _LANES)][0], _SH_HALF) & 1) * dim
            ttf = tt_all[pl.ds(coff + t, _LANES)][0].astype(jnp.float32)
            e = []
            for i in range(nblk):
                w = rows_b[t, pl.ds(off + i * _LANES, _LANES)]
                p = pos_v[s, pl.ds(i * _LANES, _LANES)]
                e.append((w + p) + (seg0[i] + ttf * segd[i]))
            s1 = (e[0] + e[1]) + (e[2] + e[3])
            s2 = (e[0] * e[0] + e[1] * e[1]) + (e[2] * e[2] + e[3] * e[3])
            mean = jnp.sum(s1) * inv_dim
            x = jnp.sum(s2) * inv_dim - mean * mean + 1e-12
            ib = 0x5F3759DF - lax.shift_right_logical(
                lax.bitcast_convert_type(x, jnp.int32), 1)
            y = lax.bitcast_convert_type(ib, jnp.float32)
            for _ in range(3):
                y = y * (1.5 - 0.5 * x * y * y)
            for i in range(nblk):
                out_b[t, pl.ds(i * _LANES, _LANES)] = (
                    (e[i] - mean) * (y * gam[i]) + bet[i])

        build_idx(0, 0)
        pltpu.async_copy(tok_hbm.at[idx[0]], rows[0], semg[0]).wait()

        def group_body(g, _):
            for b in (0, 1):
                c = 2 * g + b
                nb = 1 - b
                base = w_base + c * _CHUNK
                have_next = (g < n_groups - 1) if b else True

                def fire_next():
                    build_idx(c + 1, nb)
                    pltpu.async_copy(tok_hbm.at[idx[nb]], rows[nb], semg[nb])

                if b == 0:
                    fire_next()
                else:
                    pl.when(g < n_groups - 1)(fire_next)

                @pl.when(g >= 1)
                def _():
                    pltpu.make_async_copy(
                        outv[b],
                        out_hbm.at[pl.ds(base - 2 * _CHUNK, _CHUNK)],
                        semo[b]).wait()

                plsc.parallel_loop(0, _CHUNK, unroll=8)(
                    functools.partial(tok_body, c, rows[b], outv[b]))
                pltpu.async_copy(outv[b], out_hbm.at[pl.ds(base, _CHUNK)],
                                 semo[b])

                def wait_next():
                    pltpu.make_async_copy(tok_hbm.at[idx[nb]], rows[nb],
                                          semg[nb]).wait()

                if b == 0:
                    wait_next()
                else:
                    pl.when(g < n_groups - 1)(wait_next)
            return 0

        lax.fori_loop(0, n_groups, group_body, 0)
        pltpu.make_async_copy(
            outv[0], out_hbm.at[pl.ds(w_base + (n_chunks - 2) * _CHUNK, _CHUNK)],
            semo[0]).wait()
        pltpu.make_async_copy(
            outv[1], out_hbm.at[pl.ds(w_base + (n_chunks - 1) * _CHUNK, _CHUNK)],
            semo[1]).wait()

    return pl.kernel(
        body,
        out_type=jax.ShapeDtypeStruct((n_tokens, dim), jnp.float32),
        mesh=mesh,
        compiler_params=pltpu.CompilerParams(needs_layout_passes=False),
        scratch_types=[
            pltpu.VMEM((seq_len, dim), jnp.float32),
            pltpu.VMEM((2, dim), jnp.float32),
            pltpu.VMEM((dim,), jnp.float32),
            pltpu.VMEM((dim,), jnp.float32),
            pltpu.VMEM((per_worker + _LANES,), jnp.int32),
            pltpu.VMEM((per_worker + _LANES,), jnp.int32),
            pltpu.VMEM((_CHUNK,), jnp.int32),
            pltpu.VMEM((_CHUNK,), jnp.int32),
            pltpu.VMEM((_CHUNK, 2 * dim), jnp.float32),
            pltpu.VMEM((_CHUNK, 2 * dim), jnp.float32),
            pltpu.VMEM((_CHUNK, dim), jnp.float32),
            pltpu.VMEM((_CHUNK, dim), jnp.float32),
            pltpu.SemaphoreType.DMA,
            pltpu.SemaphoreType.DMA,
            pltpu.SemaphoreType.DMA,
            pltpu.SemaphoreType.DMA,
        ],
    )


@functools.lru_cache(maxsize=None)
def _build_tc_repack(vocab: int, dim: int):
    w = _REPACK_W
    n_blocks = (vocab + w - 1) // w

    def tkernel(x_ref, o_ref):
        x = x_ref[...]
        xt = jnp.swapaxes(x, 0, 1)
        d = x.shape[0]
        o_ref[:, 0:d] = xt[0:w // 2]
        o_ref[:, d:2 * d] = xt[w // 2:w]

    return pl.pallas_call(
        tkernel,
        grid=(n_blocks,),
        in_specs=[pl.BlockSpec((dim, w), lambda i: (0, i))],
        out_specs=pl.BlockSpec((w // 2, 2 * dim), lambda i: (i, 0)),
        out_shape=jax.ShapeDtypeStruct((n_blocks * (w // 2), 2 * dim),
                                       jnp.float32),
    )


def kernel(input_ids, token_type_ids, token_table, pos_table, seg_table,
           ln_gamma, ln_beta):
    bsz, seq = input_ids.shape
    vocab, dim = token_table.shape
    n = bsz * seq
    ids = input_ids.reshape(n).astype(jnp.int32)
    tts = token_type_ids.reshape(n).astype(jnp.int32)
    tok2 = _build_tc_repack(vocab, dim)(token_table.T)
    out = _build(n, seq, dim)(ids, tts, tok2, pos_table[:seq],
                              seg_table, ln_gamma, ln_beta)
    return out.reshape(bsz, seq, dim)
